# Initial kernel scaffold; baseline (speedup 1.0000x reference)
import functools

import jax
import jax.numpy as jnp
from jax import lax
from jax.experimental import pallas as pl
from jax.experimental.pallas import tpu as pltpu
from jax.experimental.pallas import tpu_sc as plsc

_D = 128
_LANES = 16


def _matmul_body(x_ref, w_ref, b_ref, o_ref):
    o_ref[...] = (
        jnp.dot(x_ref[...], w_ref[...], preferred_element_type=jnp.float32)
        + b_ref[...]
    )


def _transform(x, W, b):
    n = x.shape[0]
    blk = 2000 if n % 2000 == 0 else n
    grid = n // blk
    return pl.pallas_call(
        _matmul_body,
        grid=(grid,),
        in_specs=[
            pl.BlockSpec((blk, _D), lambda i: (i, 0)),
            pl.BlockSpec((_D, _D), lambda i: (0, 0)),
            pl.BlockSpec((1, _D), lambda i: (0, 0)),
        ],
        out_specs=pl.BlockSpec((blk, _D), lambda i: (i, 0)),
        out_shape=jax.ShapeDtypeStruct((n, _D), jnp.float32),
    )(x, W, b.reshape(1, _D))


def _edge_dot(h, src, dst, interpret=False):
    e_total = src.shape[0]
    info = plsc.get_sparse_core_info()
    nw = info.num_cores * info.num_subcores
    per_w = e_total // nw
    chunk = 80 if per_w % 80 == 0 else per_w
    nchunk = per_w // chunk
    mesh = plsc.VectorSubcoreMesh(core_axis_name="c", subcore_axis_name="s")

    @functools.partial(
        pl.kernel,
        out_type=jax.ShapeDtypeStruct((e_total,), jnp.float32),
        mesh=mesh,
        interpret=interpret,
        scratch_types=[
            pltpu.VMEM((chunk,), jnp.int32),
            pltpu.VMEM((chunk,), jnp.int32),
            pltpu.VMEM((chunk, _D), jnp.float32),
            pltpu.VMEM((chunk, _D), jnp.float32),
            pltpu.VMEM((chunk,), jnp.float32),
            pltpu.SemaphoreType.DMA,
            pltpu.SemaphoreType.DMA,
        ],
    )
    def k(h_hbm, src_hbm, dst_hbm, out_hbm, ia, ib, ra, rb, ov, sa, sb):
        wid = lax.axis_index("s") * info.num_cores + lax.axis_index("c")
        base = wid * per_w

        def chunk_body(c, _):
            off = base + c * chunk
            pltpu.sync_copy(src_hbm.at[pl.ds(off, chunk)], ia)
            pltpu.sync_copy(dst_hbm.at[pl.ds(off, chunk)], ib)
            ca = pltpu.async_copy(h_hbm.at[ia], ra, sa)
            cb = pltpu.async_copy(h_hbm.at[ib], rb, sb)
            ca.wait()
            cb.wait()

            def edge_body(e, _):
                acc = ra[e, pl.ds(0, _LANES)] * rb[e, pl.ds(0, _LANES)]
                for j in range(1, _D // _LANES):
                    acc = acc + (
                        ra[e, pl.ds(j * _LANES, _LANES)]
                        * rb[e, pl.ds(j * _LANES, _LANES)]
                    )
                ov[e] = jnp.sum(acc)
                return 0

            lax.fori_loop(0, chunk, edge_body, 0)
            pltpu.sync_copy(ov, out_hbm.at[pl.ds(off, chunk)])
            return 0

        lax.fori_loop(0, nchunk, chunk_body, 0)

    return k(h, src, dst)


def kernel(x, edge_index_labeled, edge_label, W, b):
    h = _transform(x, W, b)
    src = edge_index_labeled[0]
    dst = edge_index_labeled[1]
    pred = _edge_dot(h, src, dst)
    return (pred, edge_label)


# speedup vs baseline: 3.1464x; 3.1464x over previous
import functools

import jax
import jax.numpy as jnp
from jax import lax
from jax.experimental import pallas as pl
from jax.experimental.pallas import tpu as pltpu
from jax.experimental.pallas import tpu_sc as plsc

_D = 128
_LANES = 16


def _matmul_body(x_ref, w_ref, b_ref, o_ref):
    o_ref[...] = (
        jnp.dot(x_ref[...], w_ref[...], preferred_element_type=jnp.float32)
        + b_ref[...]
    )


def _transform(x, W, b):
    n = x.shape[0]
    blk = 2000 if n % 2000 == 0 else n
    grid = n // blk
    return pl.pallas_call(
        _matmul_body,
        grid=(grid,),
        in_specs=[
            pl.BlockSpec((blk, _D), lambda i: (i, 0)),
            pl.BlockSpec((_D, _D), lambda i: (0, 0)),
            pl.BlockSpec((1, _D), lambda i: (0, 0)),
        ],
        out_specs=pl.BlockSpec((blk, _D), lambda i: (i, 0)),
        out_shape=jax.ShapeDtypeStruct((n, _D), jnp.float32),
    )(x, W, b.reshape(1, _D))


def _edge_dot(h, src, dst, interpret=False):
    e_total = src.shape[0]
    ncores, nsub = 2, 16
    nw = ncores * nsub
    per_w = e_total // nw
    chunk = 80 if per_w % 80 == 0 else per_w
    nchunk = per_w // chunk
    mesh = plsc.VectorSubcoreMesh(
        core_axis_name="c", subcore_axis_name="s",
        num_cores=ncores, num_subcores=nsub,
    )

    @functools.partial(
        pl.kernel,
        out_type=jax.ShapeDtypeStruct((e_total,), jnp.float32),
        mesh=mesh,
        interpret=interpret,
        compiler_params=pltpu.CompilerParams(needs_layout_passes=False),
        scratch_types=[
            pltpu.VMEM((chunk,), jnp.int32),
            pltpu.VMEM((chunk,), jnp.int32),
            pltpu.VMEM((chunk, _D), jnp.float32),
            pltpu.VMEM((chunk, _D), jnp.float32),
            pltpu.VMEM((chunk,), jnp.float32),
            pltpu.VMEM((_LANES, _LANES), jnp.float32),
            pltpu.SemaphoreType.DMA,
            pltpu.SemaphoreType.DMA,
        ],
    )
    def k(h_hbm, src_hbm, dst_hbm, out_hbm, ia, ib, ra, rb, ov, pv, sa, sb):
        wid = lax.axis_index("s") * ncores + lax.axis_index("c")
        base = wid * per_w

        def chunk_body(c, _):
            off = base + c * chunk
            pltpu.sync_copy(src_hbm.at[pl.ds(off, chunk)], ia)
            pltpu.sync_copy(dst_hbm.at[pl.ds(off, chunk)], ib)
            ca = pltpu.async_copy(h_hbm.at[ia], ra, sa)
            cb = pltpu.async_copy(h_hbm.at[ib], rb, sb)
            ca.wait()
            cb.wait()

            lane = lax.iota(jnp.int32, _LANES)

            def group_body(g, _):
                e0 = g * _LANES
                for l in range(_LANES):
                    e = e0 + l
                    acc = ra[e, pl.ds(0, _LANES)] * rb[e, pl.ds(0, _LANES)]
                    for j in range(1, _D // _LANES):
                        acc = acc + (
                            ra[e, pl.ds(j * _LANES, _LANES)]
                            * rb[e, pl.ds(j * _LANES, _LANES)]
                        )
                    plsc.store_scatter(
                        pv, [lane, jnp.full((_LANES,), l, jnp.int32)], acc
                    )
                tot = pv[0, :]
                for i in range(1, _LANES):
                    tot = tot + pv[i, :]
                ov[pl.ds(e0, _LANES)] = tot
                return 0

            lax.fori_loop(0, chunk // _LANES, group_body, 0)
            pltpu.sync_copy(ov, out_hbm.at[pl.ds(off, chunk)])
            return 0

        lax.fori_loop(0, nchunk, chunk_body, 0)

    return k(h, src, dst)


def kernel(x, edge_index_labeled, edge_label, W, b):
    h = _transform(x, W, b)
    src = edge_index_labeled[0]
    dst = edge_index_labeled[1]
    pred = _edge_dot(h, src, dst)
    return (pred, edge_label)


# speedup vs baseline: 6.4108x; 2.0375x over previous
import functools

import jax
import jax.numpy as jnp
from jax import lax
from jax.experimental import pallas as pl
from jax.experimental.pallas import tpu as pltpu
from jax.experimental.pallas import tpu_sc as plsc

_D = 128
_LANES = 16
_CHUNK = 80


def _matmul_body(x_ref, w_ref, b_ref, o_ref):
    o_ref[...] = (
        jnp.dot(x_ref[...], w_ref[...], preferred_element_type=jnp.float32)
        + b_ref[...]
    )


def _transform(x, W, b):
    n = x.shape[0]
    blk = 2000 if n % 2000 == 0 else n
    grid = n // blk
    return pl.pallas_call(
        _matmul_body,
        grid=(grid,),
        in_specs=[
            pl.BlockSpec((blk, _D), lambda i: (i, 0)),
            pl.BlockSpec((_D, _D), lambda i: (0, 0)),
            pl.BlockSpec((1, _D), lambda i: (0, 0)),
        ],
        out_specs=pl.BlockSpec((blk, _D), lambda i: (i, 0)),
        out_shape=jax.ShapeDtypeStruct((n, _D), jnp.float32),
    )(x, W, b.reshape(1, _D))


def _edge_dot(h, src, dst, interpret=False):
    e_total = src.shape[0]
    ncores, nsub = 2, 16
    nw = ncores * nsub
    per_w = e_total // nw
    chunk = _CHUNK if per_w % _CHUNK == 0 else per_w
    nchunk = per_w // chunk
    mesh = plsc.VectorSubcoreMesh(
        core_axis_name="c", subcore_axis_name="s",
        num_cores=ncores, num_subcores=nsub,
    )

    @functools.partial(
        pl.kernel,
        out_type=jax.ShapeDtypeStruct((e_total,), jnp.float32),
        mesh=mesh,
        interpret=interpret,
        compiler_params=pltpu.CompilerParams(needs_layout_passes=False),
        scratch_types=[
            pltpu.VMEM((per_w,), jnp.int32),
            pltpu.VMEM((per_w,), jnp.int32),
            pltpu.VMEM((per_w,), jnp.float32),
            pltpu.VMEM((2, chunk, _D), jnp.float32),
            pltpu.VMEM((2, chunk, _D), jnp.float32),
            pltpu.VMEM((_LANES, _LANES), jnp.float32),
            pltpu.SemaphoreType.DMA,
            pltpu.SemaphoreType.DMA,
        ],
    )
    def k(h_hbm, src_hbm, dst_hbm, out_hbm, ia, ib, ov, ra, rb, pv, s0, s1):
        wid = lax.axis_index("s") * ncores + lax.axis_index("c")
        base = wid * per_w
        pltpu.sync_copy(src_hbm.at[pl.ds(base, per_w)], ia)
        pltpu.sync_copy(dst_hbm.at[pl.ds(base, per_w)], ib)

        sems = (s0, s1)
        lane = lax.iota(jnp.int32, _LANES)

        def issue(c, slot):
            off = c * chunk
            a = pltpu.async_copy(
                h_hbm.at[ia.at[pl.ds(off, chunk)]], ra.at[slot], sems[slot]
            )
            b = pltpu.async_copy(
                h_hbm.at[ib.at[pl.ds(off, chunk)]], rb.at[slot], sems[slot]
            )
            return a, b

        def drain(c, slot):
            off = c * chunk
            pltpu.make_async_copy(
                h_hbm.at[ia.at[pl.ds(off, chunk)]], ra.at[slot], sems[slot]
            ).wait()
            pltpu.make_async_copy(
                h_hbm.at[ib.at[pl.ds(off, chunk)]], rb.at[slot], sems[slot]
            ).wait()

        def compute(c, slot):
            def group_body(g, _):
                e0 = g * _LANES
                for l in range(_LANES):
                    e = e0 + l
                    acc = (
                        ra[slot, e, pl.ds(0, _LANES)]
                        * rb[slot, e, pl.ds(0, _LANES)]
                    )
                    for j in range(1, _D // _LANES):
                        acc = acc + (
                            ra[slot, e, pl.ds(j * _LANES, _LANES)]
                            * rb[slot, e, pl.ds(j * _LANES, _LANES)]
                        )
                    plsc.store_scatter(
                        pv, [lane, jnp.full((_LANES,), l, jnp.int32)], acc
                    )
                tot = pv[0, :]
                for i in range(1, _LANES):
                    tot = tot + pv[i, :]
                ov[pl.ds(c * chunk + e0, _LANES)] = tot
                return 0

            lax.fori_loop(0, chunk // _LANES, group_body, 0)

        issue(0, 0)
        if nchunk > 1:
            issue(1, 1)

        def pair_body(c0):
            for half in range(2):
                c = c0 + half
                drain(c, half)
                compute(c, half)

                @pl.when(c + 2 < nchunk)
                def _():
                    issue(c + 2, half)

        if nchunk > 1:
            pl.loop(0, nchunk - (nchunk % 2), step=2)(pair_body)
        if nchunk % 2 == 1 and nchunk > 1:
            drain(nchunk - 1, 0)
            compute(nchunk - 1, 0)
        if nchunk == 1:
            drain(0, 0)
            compute(0, 0)

        pltpu.sync_copy(ov, out_hbm.at[pl.ds(base, per_w)])

    return k(h, src, dst)


def kernel(x, edge_index_labeled, edge_label, W, b):
    h = _transform(x, W, b)
    src = edge_index_labeled[0]
    dst = edge_index_labeled[1]
    pred = _edge_dot(h, src, dst)
    return (pred, edge_label)


# speedup vs baseline: 7.1150x; 1.1099x over previous
import functools

import jax
import jax.numpy as jnp
from jax import lax
from jax.experimental import pallas as pl
from jax.experimental.pallas import tpu as pltpu
from jax.experimental.pallas import tpu_sc as plsc

_D = 128
_LANES = 16
_CHUNK = 80


def _matmul_body(x_ref, w_ref, b_ref, o_ref):
    o_ref[...] = (
        jnp.dot(x_ref[...], w_ref[...], preferred_element_type=jnp.float32)
        + b_ref[...]
    ).astype(jnp.bfloat16)


def _transform(x, W, b):
    n = x.shape[0]
    blk = 2000 if n % 2000 == 0 else n
    grid = n // blk
    return pl.pallas_call(
        _matmul_body,
        grid=(grid,),
        in_specs=[
            pl.BlockSpec((blk, _D), lambda i: (i, 0)),
            pl.BlockSpec((_D, _D), lambda i: (0, 0)),
            pl.BlockSpec((1, _D), lambda i: (0, 0)),
        ],
        out_specs=pl.BlockSpec((blk, _D), lambda i: (i, 0)),
        out_shape=jax.ShapeDtypeStruct((n, _D), jnp.bfloat16),
    )(x, W, b.reshape(1, _D))


def _edge_dot(h, src, dst, interpret=False):
    e_total = src.shape[0]
    ncores, nsub = 2, 16
    nw = ncores * nsub
    per_w = e_total // nw
    chunk = _CHUNK if per_w % _CHUNK == 0 else per_w
    nchunk = per_w // chunk
    mesh = plsc.VectorSubcoreMesh(
        core_axis_name="c", subcore_axis_name="s",
        num_cores=ncores, num_subcores=nsub,
    )

    @functools.partial(
        pl.kernel,
        out_type=jax.ShapeDtypeStruct((e_total,), jnp.float32),
        mesh=mesh,
        interpret=interpret,
        compiler_params=pltpu.CompilerParams(
            needs_layout_passes=False, use_tc_tiling_on_sc=False
        ),
        scratch_types=[
            pltpu.VMEM((per_w,), jnp.int32),
            pltpu.VMEM((per_w,), jnp.int32),
            pltpu.VMEM((per_w,), jnp.float32),
            pltpu.VMEM((2, chunk, _D // 2), jnp.int32),
            pltpu.VMEM((2, chunk, _D // 2), jnp.int32),
            pltpu.VMEM((_LANES, _LANES), jnp.float32),
            pltpu.SemaphoreType.DMA,
            pltpu.SemaphoreType.DMA,
        ],
    )
    def k(h_hbm, src_hbm, dst_hbm, out_hbm, ia, ib, ov, ra, rb, pv, s0, s1):
        wid = lax.axis_index("s") * ncores + lax.axis_index("c")
        base = wid * per_w
        pltpu.sync_copy(src_hbm.at[pl.ds(base, per_w)], ia)
        pltpu.sync_copy(dst_hbm.at[pl.ds(base, per_w)], ib)

        sems = (s0, s1)
        lane = lax.iota(jnp.int32, _LANES)

        def issue(c, slot):
            off = c * chunk
            a = pltpu.async_copy(
                h_hbm.at[ia.at[pl.ds(off, chunk)]], ra.at[slot], sems[slot]
            )
            b = pltpu.async_copy(
                h_hbm.at[ib.at[pl.ds(off, chunk)]], rb.at[slot], sems[slot]
            )
            return a, b

        def drain(c, slot):
            off = c * chunk
            pltpu.make_async_copy(
                h_hbm.at[ia.at[pl.ds(off, chunk)]], ra.at[slot], sems[slot]
            ).wait()
            pltpu.make_async_copy(
                h_hbm.at[ib.at[pl.ds(off, chunk)]], rb.at[slot], sems[slot]
            ).wait()

        def compute(c, slot):
            def group_body(g, _):
                e0 = g * _LANES
                for l in range(_LANES):
                    e = e0 + l
                    acc = jnp.zeros((_LANES,), jnp.float32)
                    for j in range(_D // (2 * _LANES)):
                        va = plsc.bitcast(
                            ra[slot, e, pl.ds(j * _LANES, _LANES)],
                            jnp.bfloat16,
                        )
                        vb = plsc.bitcast(
                            rb[slot, e, pl.ds(j * _LANES, _LANES)],
                            jnp.bfloat16,
                        )
                        a1, a2 = plsc.unpack(
                            va, format=plsc.PackFormat.INTERLEAVED
                        )
                        b1, b2 = plsc.unpack(
                            vb, format=plsc.PackFormat.INTERLEAVED
                        )
                        acc = acc + a1 * b1
                        acc = acc + a2 * b2
                    plsc.store_scatter(
                        pv, [lane, jnp.full((_LANES,), l, jnp.int32)], acc
                    )
                tot = pv[0, :]
                for i in range(1, _LANES):
                    tot = tot + pv[i, :]
                ov[pl.ds(c * chunk + e0, _LANES)] = tot
                return 0

            lax.fori_loop(0, chunk // _LANES, group_body, 0)

        issue(0, 0)
        if nchunk > 1:
            issue(1, 1)

        def pair_body(c0):
            for half in range(2):
                c = c0 + half
                drain(c, half)
                compute(c, half)

                @pl.when(c + 2 < nchunk)
                def _():
                    issue(c + 2, half)

        if nchunk > 1:
            pl.loop(0, nchunk - (nchunk % 2), step=2)(pair_body)
        if nchunk % 2 == 1 and nchunk > 1:
            drain(nchunk - 1, 0)
            compute(nchunk - 1, 0)
        if nchunk == 1:
            drain(0, 0)
            compute(0, 0)

        pltpu.sync_copy(ov, out_hbm.at[pl.ds(base, per_w)])

    return k(h, src, dst)


def kernel(x, edge_index_labeled, edge_label, W, b):
    h = _transform(x, W, b)
    n = h.shape[0]
    h32 = lax.bitcast_convert_type(h.reshape(n, _D // 2, 2), jnp.int32)
    src = edge_index_labeled[0]
    dst = edge_index_labeled[1]
    pred = _edge_dot(h32, src, dst)
    return (pred, edge_label)


# speedup vs baseline: 8.0064x; 1.1253x over previous
import functools

import jax
import jax.numpy as jnp
from jax import lax
from jax.experimental import pallas as pl
from jax.experimental.pallas import tpu as pltpu
from jax.experimental.pallas import tpu_sc as plsc

_D = 128
_LANES = 16
_CHUNK = 80


def _matmul_body(x_ref, w_ref, b_ref, o_ref):
    o_ref[...] = (
        jnp.dot(x_ref[...], w_ref[...], preferred_element_type=jnp.float32)
        + b_ref[...]
    ).astype(jnp.bfloat16)


def _transform(x, W, b):
    n = x.shape[0]
    blk = 2000 if n % 2000 == 0 else n
    grid = n // blk
    return pl.pallas_call(
        _matmul_body,
        grid=(grid,),
        in_specs=[
            pl.BlockSpec((blk, _D), lambda i: (i, 0)),
            pl.BlockSpec((_D, _D), lambda i: (0, 0)),
            pl.BlockSpec((1, _D), lambda i: (0, 0)),
        ],
        out_specs=pl.BlockSpec((blk, _D), lambda i: (i, 0)),
        out_shape=jax.ShapeDtypeStruct((n, _D), jnp.bfloat16),
    )(x, W, b.reshape(1, _D))


def _edge_dot(h, src, dst, interpret=False):
    e_total = src.shape[0]
    ncores, nsub = 2, 16
    nw = ncores * nsub
    per_w = e_total // nw
    chunk = _CHUNK if per_w % _CHUNK == 0 else per_w
    nchunk = per_w // chunk
    mesh = plsc.VectorSubcoreMesh(
        core_axis_name="c", subcore_axis_name="s",
        num_cores=ncores, num_subcores=nsub,
    )

    @functools.partial(
        pl.kernel,
        out_type=jax.ShapeDtypeStruct((e_total,), jnp.float32),
        mesh=mesh,
        interpret=interpret,
        compiler_params=pltpu.CompilerParams(
            needs_layout_passes=False, use_tc_tiling_on_sc=False
        ),
        scratch_types=[
            pltpu.VMEM((per_w,), jnp.int32),
            pltpu.VMEM((per_w,), jnp.int32),
            pltpu.VMEM((per_w,), jnp.float32),
            pltpu.VMEM((2, chunk, _D // 2), jnp.int32),
            pltpu.VMEM((2, chunk, _D // 2), jnp.int32),
            pltpu.VMEM((_LANES, _LANES), jnp.float32),
            pltpu.SemaphoreType.DMA,
            pltpu.SemaphoreType.DMA,
        ],
    )
    def k(h_hbm, src_hbm, dst_hbm, out_hbm, ia, ib, ov, ra, rb, pv, s0, s1):
        wid = lax.axis_index("s") * ncores + lax.axis_index("c")
        base = wid * per_w
        pltpu.sync_copy(src_hbm.at[pl.ds(base, per_w)], ia)
        pltpu.sync_copy(dst_hbm.at[pl.ds(base, per_w)], ib)

        sems = (s0, s1)
        lane = lax.iota(jnp.int32, _LANES)

        def issue(c, slot):
            off = c * chunk
            a = pltpu.async_copy(
                h_hbm.at[ia.at[pl.ds(off, chunk)]], ra.at[slot], sems[slot]
            )
            b = pltpu.async_copy(
                h_hbm.at[ib.at[pl.ds(off, chunk)]], rb.at[slot], sems[slot]
            )
            return a, b

        def drain(c, slot):
            off = c * chunk
            pltpu.make_async_copy(
                h_hbm.at[ia.at[pl.ds(off, chunk)]], ra.at[slot], sems[slot]
            ).wait()
            pltpu.make_async_copy(
                h_hbm.at[ib.at[pl.ds(off, chunk)]], rb.at[slot], sems[slot]
            ).wait()

        def compute(c, slot):
            def group_body(g, _):
                e0 = g * _LANES
                for l in range(_LANES):
                    e = e0 + l
                    acc1 = acc2 = None
                    for j in range(_D // (2 * _LANES)):
                        va = plsc.bitcast(
                            ra[slot, e, pl.ds(j * _LANES, _LANES)],
                            jnp.bfloat16,
                        )
                        vb = plsc.bitcast(
                            rb[slot, e, pl.ds(j * _LANES, _LANES)],
                            jnp.bfloat16,
                        )
                        p1, p2 = plsc.unpack(
                            va * vb, format=plsc.PackFormat.INTERLEAVED
                        )
                        if acc1 is None:
                            acc1, acc2 = p1, p2
                        else:
                            acc1 = acc1 + p1
                            acc2 = acc2 + p2
                    plsc.store_scatter(
                        pv,
                        [lane, jnp.full((_LANES,), l, jnp.int32)],
                        acc1 + acc2,
                    )
                tot = pv[0, :]
                for i in range(1, _LANES):
                    tot = tot + pv[i, :]
                ov[pl.ds(c * chunk + e0, _LANES)] = tot
                return 0

            lax.fori_loop(0, chunk // _LANES, group_body, 0)

        issue(0, 0)
        if nchunk > 1:
            issue(1, 1)

        def pair_body(c0):
            for half in range(2):
                c = c0 + half
                drain(c, half)
                compute(c, half)

                @pl.when(c + 2 < nchunk)
                def _():
                    issue(c + 2, half)

        if nchunk > 1:
            pl.loop(0, nchunk - (nchunk % 2), step=2)(pair_body)
        if nchunk % 2 == 1 and nchunk > 1:
            drain(nchunk - 1, 0)
            compute(nchunk - 1, 0)
        if nchunk == 1:
            drain(0, 0)
            compute(0, 0)

        pltpu.sync_copy(ov, out_hbm.at[pl.ds(base, per_w)])

    return k(h, src, dst)


def kernel(x, edge_index_labeled, edge_label, W, b):
    h = _transform(x, W, b)
    n = h.shape[0]
    h32 = lax.bitcast_convert_type(h.reshape(n, _D // 2, 2), jnp.int32)
    src = edge_index_labeled[0]
    dst = edge_index_labeled[1]
    pred = _edge_dot(h32, src, dst)
    return (pred, edge_label)


# speedup vs baseline: 8.9562x; 1.1186x over previous
import functools

import jax
import jax.numpy as jnp
from jax import lax
from jax.experimental import pallas as pl
from jax.experimental.pallas import tpu as pltpu
from jax.experimental.pallas import tpu_sc as plsc

_D = 128
_LANES = 16
_CHUNK = 80
_BLK = 4


def _matmul_body(x_ref, w_ref, b_ref, o_ref):
    o_ref[...] = (
        jnp.dot(x_ref[...], w_ref[...], preferred_element_type=jnp.float32)
        + b_ref[...]
    ).astype(jnp.bfloat16)


def _transform(x, W, b):
    n = x.shape[0]
    blk = 2000 if n % 2000 == 0 else n
    grid = n // blk
    return pl.pallas_call(
        _matmul_body,
        grid=(grid,),
        in_specs=[
            pl.BlockSpec((blk, _D), lambda i: (i, 0)),
            pl.BlockSpec((_D, _D), lambda i: (0, 0)),
            pl.BlockSpec((1, _D), lambda i: (0, 0)),
        ],
        out_specs=pl.BlockSpec((blk, _D), lambda i: (i, 0)),
        out_shape=jax.ShapeDtypeStruct((n, _D), jnp.bfloat16),
    )(x, W, b.reshape(1, _D))


def _edge_dot(h, src, dst, interpret=False):
    e_total = src.shape[0]
    ncores, nsub = 2, 16
    nw = ncores * nsub
    per_w = e_total // nw
    chunk = _CHUNK if per_w % _CHUNK == 0 else per_w
    nchunk = per_w // chunk
    mesh = plsc.VectorSubcoreMesh(
        core_axis_name="c", subcore_axis_name="s",
        num_cores=ncores, num_subcores=nsub,
    )

    @functools.partial(
        pl.kernel,
        out_type=jax.ShapeDtypeStruct((e_total,), jnp.float32),
        mesh=mesh,
        interpret=interpret,
        compiler_params=pltpu.CompilerParams(
            needs_layout_passes=False, use_tc_tiling_on_sc=False
        ),
        scratch_types=[
            pltpu.VMEM((per_w,), jnp.int32),
            pltpu.VMEM((per_w,), jnp.int32),
            pltpu.VMEM((per_w,), jnp.float32),
            pltpu.VMEM((2, chunk, _D // 2), jnp.int32),
            pltpu.VMEM((2, chunk, _D // 2), jnp.int32),
            pltpu.VMEM((_LANES, _LANES), jnp.float32),
            pltpu.SemaphoreType.DMA,
            pltpu.SemaphoreType.DMA,
        ],
    )
    def k(h_hbm, src_hbm, dst_hbm, out_hbm, ia, ib, ov, ra, rb, pv, s0, s1):
        wid = lax.axis_index("s") * ncores + lax.axis_index("c")
        base = wid * per_w
        pltpu.sync_copy(src_hbm.at[pl.ds(base, per_w)], ia)
        pltpu.sync_copy(dst_hbm.at[pl.ds(base, per_w)], ib)

        sems = (s0, s1)
        lane = lax.iota(jnp.int32, _LANES)

        def issue(c, slot):
            off = c * chunk
            a = pltpu.async_copy(
                h_hbm.at[ia.at[pl.ds(off, chunk)]], ra.at[slot], sems[slot]
            )
            b = pltpu.async_copy(
                h_hbm.at[ib.at[pl.ds(off, chunk)]], rb.at[slot], sems[slot]
            )
            return a, b

        def drain(c, slot):
            off = c * chunk
            pltpu.make_async_copy(
                h_hbm.at[ia.at[pl.ds(off, chunk)]], ra.at[slot], sems[slot]
            ).wait()
            pltpu.make_async_copy(
                h_hbm.at[ib.at[pl.ds(off, chunk)]], rb.at[slot], sems[slot]
            ).wait()

        def compute(c, slot):
            def group_body(g, _):
                e0 = g * _LANES
                nj = _D // (2 * _LANES)
                for blk in range(0, _LANES, _BLK):
                    va = {}
                    vb = {}
                    for l in range(blk, blk + _BLK):
                        e = e0 + l
                        for j in range(nj):
                            va[l, j] = plsc.bitcast(
                                ra[slot, e, pl.ds(j * _LANES, _LANES)],
                                jnp.bfloat16,
                            )
                            vb[l, j] = plsc.bitcast(
                                rb[slot, e, pl.ds(j * _LANES, _LANES)],
                                jnp.bfloat16,
                            )
                    outs = []
                    for l in range(blk, blk + _BLK):
                        acc1 = acc2 = None
                        for j in range(nj):
                            p1, p2 = plsc.unpack(
                                va[l, j] * vb[l, j],
                                format=plsc.PackFormat.INTERLEAVED,
                            )
                            if acc1 is None:
                                acc1, acc2 = p1, p2
                            else:
                                acc1 = acc1 + p1
                                acc2 = acc2 + p2
                        outs.append((l, acc1 + acc2))
                    for l, acc in outs:
                        plsc.store_scatter(
                            pv,
                            [lane, jnp.full((_LANES,), l, jnp.int32)],
                            acc,
                        )
                tot = pv[0, :]
                for i in range(1, _LANES):
                    tot = tot + pv[i, :]
                ov[pl.ds(c * chunk + e0, _LANES)] = tot
                return 0

            lax.fori_loop(0, chunk // _LANES, group_body, 0)

        issue(0, 0)
        if nchunk > 1:
            issue(1, 1)

        def pair_body(c0):
            for half in range(2):
                c = c0 + half
                drain(c, half)
                compute(c, half)

                @pl.when(c + 2 < nchunk)
                def _():
                    issue(c + 2, half)

        if nchunk > 1:
            pl.loop(0, nchunk - (nchunk % 2), step=2)(pair_body)
        if nchunk % 2 == 1 and nchunk > 1:
            drain(nchunk - 1, 0)
            compute(nchunk - 1, 0)
        if nchunk == 1:
            drain(0, 0)
            compute(0, 0)

        pltpu.sync_copy(ov, out_hbm.at[pl.ds(base, per_w)])

    return k(h, src, dst)


def kernel(x, edge_index_labeled, edge_label, W, b):
    h = _transform(x, W, b)
    n = h.shape[0]
    h32 = lax.bitcast_convert_type(h.reshape(n, _D // 2, 2), jnp.int32)
    src = edge_index_labeled[0]
    dst = edge_index_labeled[1]
    pred = _edge_dot(h32, src, dst)
    return (pred, edge_label)


# speedup vs baseline: 9.4612x; 1.0564x over previous
import functools

import jax
import jax.numpy as jnp
from jax import lax
from jax.experimental import pallas as pl
from jax.experimental.pallas import tpu as pltpu
from jax.experimental.pallas import tpu_sc as plsc

_D = 128
_LANES = 16
_CHUNK = 80
_BLK = 8


def _matmul_body(x_ref, w_ref, b_ref, o_ref):
    o_ref[...] = (
        jnp.dot(x_ref[...], w_ref[...], preferred_element_type=jnp.float32)
        + b_ref[...]
    ).astype(jnp.bfloat16)


def _transform(x, W, b):
    n = x.shape[0]
    blk = 2000 if n % 2000 == 0 else n
    grid = n // blk
    return pl.pallas_call(
        _matmul_body,
        grid=(grid,),
        in_specs=[
            pl.BlockSpec((blk, _D), lambda i: (i, 0)),
            pl.BlockSpec((_D, _D), lambda i: (0, 0)),
            pl.BlockSpec((1, _D), lambda i: (0, 0)),
        ],
        out_specs=pl.BlockSpec((blk, _D), lambda i: (i, 0)),
        out_shape=jax.ShapeDtypeStruct((n, _D), jnp.bfloat16),
    )(x, W, b.reshape(1, _D))


def _edge_dot(h, src, dst, interpret=False):
    e_total = src.shape[0]
    ncores, nsub = 2, 16
    nw = ncores * nsub
    per_w = e_total // nw
    chunk = _CHUNK if per_w % _CHUNK == 0 else per_w
    nchunk = per_w // chunk
    mesh = plsc.VectorSubcoreMesh(
        core_axis_name="c", subcore_axis_name="s",
        num_cores=ncores, num_subcores=nsub,
    )

    @functools.partial(
        pl.kernel,
        out_type=jax.ShapeDtypeStruct((e_total,), jnp.float32),
        mesh=mesh,
        interpret=interpret,
        compiler_params=pltpu.CompilerParams(
            needs_layout_passes=False, use_tc_tiling_on_sc=False
        ),
        scratch_types=[
            pltpu.VMEM((per_w,), jnp.int32),
            pltpu.VMEM((per_w,), jnp.int32),
            pltpu.VMEM((per_w,), jnp.float32),
            pltpu.VMEM((2, chunk, _D // 2), jnp.int32),
            pltpu.VMEM((2, chunk, _D // 2), jnp.int32),
            pltpu.VMEM((_LANES, _LANES), jnp.float32),
            pltpu.SemaphoreType.DMA,
            pltpu.SemaphoreType.DMA,
        ],
    )
    def k(h_hbm, src_hbm, dst_hbm, out_hbm, ia, ib, ov, ra, rb, pv, s0, s1):
        wid = lax.axis_index("s") * ncores + lax.axis_index("c")
        base = wid * per_w
        pltpu.sync_copy(src_hbm.at[pl.ds(base, per_w)], ia)
        pltpu.sync_copy(dst_hbm.at[pl.ds(base, per_w)], ib)

        sems = (s0, s1)
        lane = lax.iota(jnp.int32, _LANES)

        def issue(c, slot):
            off = c * chunk
            a = pltpu.async_copy(
                h_hbm.at[ia.at[pl.ds(off, chunk)]], ra.at[slot], sems[slot]
            )
            b = pltpu.async_copy(
                h_hbm.at[ib.at[pl.ds(off, chunk)]], rb.at[slot], sems[slot]
            )
            return a, b

        def drain(c, slot):
            off = c * chunk
            pltpu.make_async_copy(
                h_hbm.at[ia.at[pl.ds(off, chunk)]], ra.at[slot], sems[slot]
            ).wait()
            pltpu.make_async_copy(
                h_hbm.at[ib.at[pl.ds(off, chunk)]], rb.at[slot], sems[slot]
            ).wait()

        def compute(c, slot):
            def group_body(g, _):
                e0 = g * _LANES
                nj = _D // (2 * _LANES)
                for blk in range(0, _LANES, _BLK):
                    va = {}
                    vb = {}
                    for l in range(blk, blk + _BLK):
                        e = e0 + l
                        for j in range(nj):
                            va[l, j] = plsc.bitcast(
                                ra[slot, e, pl.ds(j * _LANES, _LANES)],
                                jnp.bfloat16,
                            )
                            vb[l, j] = plsc.bitcast(
                                rb[slot, e, pl.ds(j * _LANES, _LANES)],
                                jnp.bfloat16,
                            )
                    outs = []
                    for l in range(blk, blk + _BLK):
                        acc1 = acc2 = None
                        for j in range(nj):
                            p1, p2 = plsc.unpack(
                                va[l, j] * vb[l, j],
                                format=plsc.PackFormat.INTERLEAVED,
                            )
                            if acc1 is None:
                                acc1, acc2 = p1, p2
                            else:
                                acc1 = acc1 + p1
                                acc2 = acc2 + p2
                        outs.append((l, acc1 + acc2))
                    for l, acc in outs:
                        plsc.store_scatter(
                            pv,
                            [lane, jnp.full((_LANES,), l, jnp.int32)],
                            acc,
                        )
                tot = pv[0, :]
                for i in range(1, _LANES):
                    tot = tot + pv[i, :]
                ov[pl.ds(c * chunk + e0, _LANES)] = tot
                return 0

            lax.fori_loop(0, chunk // _LANES, group_body, 0)

        issue(0, 0)
        if nchunk > 1:
            issue(1, 1)

        def pair_body(c0):
            for half in range(2):
                c = c0 + half
                drain(c, half)
                compute(c, half)

                @pl.when(c + 2 < nchunk)
                def _():
                    issue(c + 2, half)

        if nchunk > 1:
            pl.loop(0, nchunk - (nchunk % 2), step=2)(pair_body)
        if nchunk % 2 == 1 and nchunk > 1:
            drain(nchunk - 1, 0)
            compute(nchunk - 1, 0)
        if nchunk == 1:
            drain(0, 0)
            compute(0, 0)

        pltpu.sync_copy(ov, out_hbm.at[pl.ds(base, per_w)])

    return k(h, src, dst)


def kernel(x, edge_index_labeled, edge_label, W, b):
    h = _transform(x, W, b)
    n = h.shape[0]
    h32 = lax.bitcast_convert_type(h.reshape(n, _D // 2, 2), jnp.int32)
    src = edge_index_labeled[0]
    dst = edge_index_labeled[1]
    pred = _edge_dot(h32, src, dst)
    return (pred, edge_label)


# speedup vs baseline: 11.2259x; 1.1865x over previous
import functools

import jax
import jax.numpy as jnp
from jax import lax
from jax.experimental import pallas as pl
from jax.experimental.pallas import tpu as pltpu
from jax.experimental.pallas import tpu_sc as plsc

_D = 128
_LANES = 16
_CHUNK = 80
_BLK = 8


def _matmul_body(x_ref, w_ref, b_ref, o_ref):
    o_ref[...] = (
        jnp.dot(x_ref[...], w_ref[...], preferred_element_type=jnp.float32)
        + b_ref[...]
    ).astype(jnp.bfloat16)


def _transform(x, W, b):
    n = x.shape[0]
    blk = 2000 if n % 2000 == 0 else n
    grid = n // blk
    return pl.pallas_call(
        _matmul_body,
        grid=(grid,),
        in_specs=[
            pl.BlockSpec((blk, _D), lambda i: (i, 0)),
            pl.BlockSpec((_D, _D), lambda i: (0, 0)),
            pl.BlockSpec((1, _D), lambda i: (0, 0)),
        ],
        out_specs=pl.BlockSpec((blk, _D), lambda i: (i, 0)),
        out_shape=jax.ShapeDtypeStruct((n, _D), jnp.bfloat16),
    )(x, W, b.reshape(1, _D))


def _edge_dot(h, src, dst, interpret=False):
    e_total = src.shape[0]
    ncores, nsub = 2, 16
    nw = ncores * nsub
    per_w = e_total // nw
    chunk = _CHUNK if per_w % _CHUNK == 0 else per_w
    nchunk = per_w // chunk
    mesh = plsc.VectorSubcoreMesh(
        core_axis_name="c", subcore_axis_name="s",
        num_cores=ncores, num_subcores=nsub,
    )

    @functools.partial(
        pl.kernel,
        out_type=jax.ShapeDtypeStruct((e_total,), jnp.float32),
        mesh=mesh,
        interpret=interpret,
        compiler_params=pltpu.CompilerParams(
            needs_layout_passes=False, use_tc_tiling_on_sc=False
        ),
        scratch_types=[
            pltpu.VMEM((per_w,), jnp.int32),
            pltpu.VMEM((per_w,), jnp.int32),
            pltpu.VMEM((per_w,), jnp.float32),
            pltpu.VMEM((2, chunk, _D // 2), jnp.int32),
            pltpu.VMEM((2, chunk, _D // 2), jnp.int32),
            pltpu.VMEM((_LANES, _LANES), jnp.float32),
            pltpu.VMEM_SHARED((10000, _D // 2), jnp.int32),
            pltpu.SemaphoreType.DMA,
            pltpu.SemaphoreType.DMA,
        ],
    )
    def k(h_hbm, src_hbm, dst_hbm, out_hbm, ia, ib, ov, ra, rb, pv, hs, s0, s1):
        wid = lax.axis_index("s") * ncores + lax.axis_index("c")
        base = wid * per_w
        n_nodes = h_hbm.shape[0]
        rows_per_sub = n_nodes // nsub
        sid = lax.axis_index("s")
        pltpu.sync_copy(
            h_hbm.at[pl.ds(sid * rows_per_sub, rows_per_sub)],
            hs.at[pl.ds(sid * rows_per_sub, rows_per_sub)],
        )
        pltpu.sync_copy(src_hbm.at[pl.ds(base, per_w)], ia)
        pltpu.sync_copy(dst_hbm.at[pl.ds(base, per_w)], ib)
        plsc.subcore_barrier()

        sems = (s0, s1)
        lane = lax.iota(jnp.int32, _LANES)

        def issue(c, slot):
            off = c * chunk
            a = pltpu.async_copy(
                hs.at[ia.at[pl.ds(off, chunk)]], ra.at[slot], sems[slot]
            )
            b = pltpu.async_copy(
                hs.at[ib.at[pl.ds(off, chunk)]], rb.at[slot], sems[slot]
            )
            return a, b

        def drain(c, slot):
            off = c * chunk
            pltpu.make_async_copy(
                hs.at[ia.at[pl.ds(off, chunk)]], ra.at[slot], sems[slot]
            ).wait()
            pltpu.make_async_copy(
                hs.at[ib.at[pl.ds(off, chunk)]], rb.at[slot], sems[slot]
            ).wait()

        def compute(c, slot):
            def group_body(g, _):
                e0 = g * _LANES
                nj = _D // (2 * _LANES)
                for blk in range(0, _LANES, _BLK):
                    va = {}
                    vb = {}
                    for l in range(blk, blk + _BLK):
                        e = e0 + l
                        for j in range(nj):
                            va[l, j] = plsc.bitcast(
                                ra[slot, e, pl.ds(j * _LANES, _LANES)],
                                jnp.bfloat16,
                            )
                            vb[l, j] = plsc.bitcast(
                                rb[slot, e, pl.ds(j * _LANES, _LANES)],
                                jnp.bfloat16,
                            )
                    outs = []
                    for l in range(blk, blk + _BLK):
                        acc1 = acc2 = None
                        for j in range(nj):
                            p1, p2 = plsc.unpack(
                                va[l, j] * vb[l, j],
                                format=plsc.PackFormat.INTERLEAVED,
                            )
                            if acc1 is None:
                                acc1, acc2 = p1, p2
                            else:
                                acc1 = acc1 + p1
                                acc2 = acc2 + p2
                        outs.append((l, acc1 + acc2))
                    for l, acc in outs:
                        plsc.store_scatter(
                            pv,
                            [lane, jnp.full((_LANES,), l, jnp.int32)],
                            acc,
                        )
                tot = pv[0, :]
                for i in range(1, _LANES):
                    tot = tot + pv[i, :]
                ov[pl.ds(c * chunk + e0, _LANES)] = tot
                return 0

            lax.fori_loop(0, chunk // _LANES, group_body, 0)

        issue(0, 0)
        if nchunk > 1:
            issue(1, 1)

        def pair_body(c0):
            for half in range(2):
                c = c0 + half
                drain(c, half)
                compute(c, half)

                @pl.when(c + 2 < nchunk)
                def _():
                    issue(c + 2, half)

        if nchunk > 1:
            pl.loop(0, nchunk - (nchunk % 2), step=2)(pair_body)
        if nchunk % 2 == 1 and nchunk > 1:
            drain(nchunk - 1, 0)
            compute(nchunk - 1, 0)
        if nchunk == 1:
            drain(0, 0)
            compute(0, 0)

        pltpu.sync_copy(ov, out_hbm.at[pl.ds(base, per_w)])

    return k(h, src, dst)


def kernel(x, edge_index_labeled, edge_label, W, b):
    h = _transform(x, W, b)
    n = h.shape[0]
    h32 = lax.bitcast_convert_type(h.reshape(n, _D // 2, 2), jnp.int32)
    src = edge_index_labeled[0]
    dst = edge_index_labeled[1]
    pred = _edge_dot(h32, src, dst)
    return (pred, edge_label)


# speedup vs baseline: 11.7025x; 1.0425x over previous
import functools

import jax
import jax.numpy as jnp
from jax import lax
from jax.experimental import pallas as pl
from jax.experimental.pallas import tpu as pltpu
from jax.experimental.pallas import tpu_sc as plsc

_D = 128
_LANES = 16
_CHUNK = 80
_BLK = 8


def _matmul_body(x_ref, w_ref, b_ref, o_ref):
    o_ref[...] = (
        jnp.dot(x_ref[...], w_ref[...], preferred_element_type=jnp.float32)
        + b_ref[...]
    ).astype(jnp.bfloat16)


def _transform(x, W, b):
    n = x.shape[0]
    blk = 2000 if n % 2000 == 0 else n
    grid = n // blk
    return pl.pallas_call(
        _matmul_body,
        grid=(grid,),
        in_specs=[
            pl.BlockSpec((blk, _D), lambda i: (i, 0)),
            pl.BlockSpec((_D, _D), lambda i: (0, 0)),
            pl.BlockSpec((1, _D), lambda i: (0, 0)),
        ],
        out_specs=pl.BlockSpec((blk, _D), lambda i: (i, 0)),
        out_shape=jax.ShapeDtypeStruct((n, _D), jnp.bfloat16),
    )(x, W, b.reshape(1, _D))


def _edge_dot(h, src, dst, interpret=False):
    e_total = src.shape[0]
    ncores, nsub = 2, 16
    nw = ncores * nsub
    per_w = e_total // nw
    chunk = _CHUNK if per_w % _CHUNK == 0 else per_w
    nchunk = per_w // chunk
    mesh = plsc.VectorSubcoreMesh(
        core_axis_name="c", subcore_axis_name="s",
        num_cores=ncores, num_subcores=nsub,
    )

    @functools.partial(
        pl.kernel,
        out_type=jax.ShapeDtypeStruct((e_total,), jnp.float32),
        mesh=mesh,
        interpret=interpret,
        compiler_params=pltpu.CompilerParams(
            needs_layout_passes=False, use_tc_tiling_on_sc=False
        ),
        scratch_types=[
            pltpu.VMEM((per_w,), jnp.int32),
            pltpu.VMEM((per_w,), jnp.int32),
            pltpu.VMEM((per_w,), jnp.float32),
            pltpu.VMEM((2, chunk, _D // 2), jnp.int32),
            pltpu.VMEM((2, chunk, _D // 2), jnp.int32),
            pltpu.VMEM((_LANES, _LANES), jnp.float32),
            pltpu.VMEM_SHARED((10000, _D // 2), jnp.int32),
            pltpu.SemaphoreType.DMA,
            pltpu.SemaphoreType.DMA,
        ],
    )
    def k(h_hbm, src_hbm, dst_hbm, out_hbm, ia, ib, ov, ra, rb, pv, hs, s0, s1):
        wid = lax.axis_index("s") * ncores + lax.axis_index("c")
        base = wid * per_w
        n_nodes = h_hbm.shape[0]
        rows_per_sub = n_nodes // nsub
        sid = lax.axis_index("s")
        pltpu.sync_copy(
            h_hbm.at[pl.ds(sid * rows_per_sub, rows_per_sub)],
            hs.at[pl.ds(sid * rows_per_sub, rows_per_sub)],
        )
        pltpu.sync_copy(src_hbm.at[pl.ds(base, per_w)], ia)
        pltpu.sync_copy(dst_hbm.at[pl.ds(base, per_w)], ib)
        plsc.subcore_barrier()

        sems = (s0, s1)
        lane = lax.iota(jnp.int32, _LANES)

        def issue(c, slot):
            off = c * chunk
            a = pltpu.async_copy(
                hs.at[ia.at[pl.ds(off, chunk)]], ra.at[slot], sems[slot]
            )
            b = pltpu.async_copy(
                hs.at[ib.at[pl.ds(off, chunk)]], rb.at[slot], sems[slot]
            )
            return a, b

        def drain(c, slot):
            off = c * chunk
            pltpu.make_async_copy(
                hs.at[ia.at[pl.ds(off, chunk)]], ra.at[slot], sems[slot]
            ).wait()
            pltpu.make_async_copy(
                hs.at[ib.at[pl.ds(off, chunk)]], rb.at[slot], sems[slot]
            ).wait()

        def compute(c, slot):
            def group_body(g, _):
                e0 = g * _LANES
                nj = _D // (2 * _LANES)
                res = jnp.zeros((_LANES,), jnp.float32)
                for l in range(_LANES):
                    e = e0 + l
                    acc1 = acc2 = None
                    for j in range(nj):
                        va = plsc.bitcast(
                            ra[slot, e, pl.ds(j * _LANES, _LANES)],
                            jnp.bfloat16,
                        )
                        vb = plsc.bitcast(
                            rb[slot, e, pl.ds(j * _LANES, _LANES)],
                            jnp.bfloat16,
                        )
                        p1, p2 = plsc.unpack(
                            va * vb, format=plsc.PackFormat.INTERLEAVED
                        )
                        if acc1 is None:
                            acc1, acc2 = p1, p2
                        else:
                            acc1 = acc1 + p1
                            acc2 = acc2 + p2
                    res = jnp.where(lane == l, jnp.sum(acc1 + acc2), res)
                ov[pl.ds(c * chunk + e0, _LANES)] = res
                return 0

            lax.fori_loop(0, chunk // _LANES, group_body, 0)

        issue(0, 0)
        if nchunk > 1:
            issue(1, 1)

        def pair_body(c0):
            for half in range(2):
                c = c0 + half
                drain(c, half)
                compute(c, half)

                @pl.when(c + 2 < nchunk)
                def _():
                    issue(c + 2, half)

        if nchunk > 1:
            pl.loop(0, nchunk - (nchunk % 2), step=2)(pair_body)
        if nchunk % 2 == 1 and nchunk > 1:
            drain(nchunk - 1, 0)
            compute(nchunk - 1, 0)
        if nchunk == 1:
            drain(0, 0)
            compute(0, 0)

        pltpu.sync_copy(ov, out_hbm.at[pl.ds(base, per_w)])

    return k(h, src, dst)


def kernel(x, edge_index_labeled, edge_label, W, b):
    h = _transform(x, W, b)
    n = h.shape[0]
    h32 = lax.bitcast_convert_type(h.reshape(n, _D // 2, 2), jnp.int32)
    src = edge_index_labeled[0]
    dst = edge_index_labeled[1]
    pred = _edge_dot(h32, src, dst)
    return (pred, edge_label)


# speedup vs baseline: 11.8473x; 1.0124x over previous
import functools

import jax
import jax.numpy as jnp
from jax import lax
from jax.experimental import pallas as pl
from jax.experimental.pallas import tpu as pltpu
from jax.experimental.pallas import tpu_sc as plsc

_D = 128
_LANES = 16
_CHUNK = 80
_BLK = 8


def _matmul_body(x_ref, w_ref, b_ref, o_ref):
    o_ref[...] = (
        jnp.dot(x_ref[...], w_ref[...], preferred_element_type=jnp.float32)
        + b_ref[...]
    ).astype(jnp.bfloat16)


def _transform(x, W, b):
    n = x.shape[0]
    blk = 2000 if n % 2000 == 0 else n
    grid = n // blk
    return pl.pallas_call(
        _matmul_body,
        grid=(grid,),
        in_specs=[
            pl.BlockSpec((blk, _D), lambda i: (i, 0)),
            pl.BlockSpec((_D, _D), lambda i: (0, 0)),
            pl.BlockSpec((1, _D), lambda i: (0, 0)),
        ],
        out_specs=pl.BlockSpec((blk, _D), lambda i: (i, 0)),
        out_shape=jax.ShapeDtypeStruct((n, _D), jnp.bfloat16),
    )(x, W, b.reshape(1, _D))


def _edge_dot(h, src, dst, interpret=False):
    e_total = src.shape[0]
    ncores, nsub = 2, 16
    nw = ncores * nsub
    per_w = e_total // nw
    chunk = _CHUNK if per_w % _CHUNK == 0 else per_w
    nchunk = per_w // chunk
    mesh = plsc.VectorSubcoreMesh(
        core_axis_name="c", subcore_axis_name="s",
        num_cores=ncores, num_subcores=nsub,
    )

    @functools.partial(
        pl.kernel,
        out_type=jax.ShapeDtypeStruct((e_total,), jnp.float32),
        mesh=mesh,
        interpret=interpret,
        compiler_params=pltpu.CompilerParams(
            needs_layout_passes=False, use_tc_tiling_on_sc=False
        ),
        scratch_types=[
            pltpu.VMEM((per_w,), jnp.int32),
            pltpu.VMEM((per_w,), jnp.int32),
            pltpu.VMEM((per_w,), jnp.float32),
            pltpu.VMEM((2, chunk, _D // 2), jnp.int32),
            pltpu.VMEM((2, chunk, _D // 2), jnp.int32),
            pltpu.VMEM((_LANES, _LANES), jnp.float32),
            pltpu.VMEM_SHARED((10000, _D // 2), jnp.int32),
            pltpu.SemaphoreType.DMA,
            pltpu.SemaphoreType.DMA,
        ],
    )
    def k(h_hbm, src_hbm, dst_hbm, out_hbm, ia, ib, ov, ra, rb, pv, hs, s0, s1):
        wid = lax.axis_index("s") * ncores + lax.axis_index("c")
        base = wid * per_w
        n_nodes = h_hbm.shape[0]
        rows_per_sub = n_nodes // nsub
        sid = lax.axis_index("s")
        pltpu.sync_copy(
            h_hbm.at[pl.ds(sid * rows_per_sub, rows_per_sub)],
            hs.at[pl.ds(sid * rows_per_sub, rows_per_sub)],
        )
        pltpu.sync_copy(src_hbm.at[pl.ds(base, per_w)], ia)
        pltpu.sync_copy(dst_hbm.at[pl.ds(base, per_w)], ib)
        plsc.subcore_barrier()

        sems = (s0, s1)
        lane = lax.iota(jnp.int32, _LANES)

        def issue(c, slot):
            off = c * chunk
            a = pltpu.async_copy(
                hs.at[ia.at[pl.ds(off, chunk)]], ra.at[slot], sems[slot]
            )
            b = pltpu.async_copy(
                hs.at[ib.at[pl.ds(off, chunk)]], rb.at[slot], sems[slot]
            )
            return a, b

        def drain(c, slot):
            off = c * chunk
            pltpu.make_async_copy(
                hs.at[ia.at[pl.ds(off, chunk)]], ra.at[slot], sems[slot]
            ).wait()
            pltpu.make_async_copy(
                hs.at[ib.at[pl.ds(off, chunk)]], rb.at[slot], sems[slot]
            ).wait()

        def compute(c, slot):
            def group_body(g, _):
                e0 = g * _LANES
                nj = _D // (2 * _LANES)
                res = jnp.zeros((_LANES,), jnp.float32)
                for l in range(_LANES):
                    e = e0 + l
                    prods = []
                    for j in range(nj):
                        va = plsc.bitcast(
                            ra[slot, e, pl.ds(j * _LANES, _LANES)],
                            jnp.bfloat16,
                        )
                        vb = plsc.bitcast(
                            rb[slot, e, pl.ds(j * _LANES, _LANES)],
                            jnp.bfloat16,
                        )
                        prods.append(va * vb)
                    q1 = prods[0] + prods[1]
                    q2 = prods[2] + prods[3]
                    u1, u2 = plsc.unpack(q1, format=plsc.PackFormat.INTERLEAVED)
                    u3, u4 = plsc.unpack(q2, format=plsc.PackFormat.INTERLEAVED)
                    res = jnp.where(
                        lane == l, jnp.sum((u1 + u2) + (u3 + u4)), res
                    )
                ov[pl.ds(c * chunk + e0, _LANES)] = res
                return 0

            lax.fori_loop(0, chunk // _LANES, group_body, 0)

        issue(0, 0)
        if nchunk > 1:
            issue(1, 1)

        def pair_body(c0):
            for half in range(2):
                c = c0 + half
                drain(c, half)
                compute(c, half)

                @pl.when(c + 2 < nchunk)
                def _():
                    issue(c + 2, half)

        if nchunk > 1:
            pl.loop(0, nchunk - (nchunk % 2), step=2)(pair_body)
        if nchunk % 2 == 1 and nchunk > 1:
            drain(nchunk - 1, 0)
            compute(nchunk - 1, 0)
        if nchunk == 1:
            drain(0, 0)
            compute(0, 0)

        pltpu.sync_copy(ov, out_hbm.at[pl.ds(base, per_w)])

    return k(h, src, dst)


def kernel(x, edge_index_labeled, edge_label, W, b):
    h = _transform(x, W, b)
    n = h.shape[0]
    h32 = lax.bitcast_convert_type(h.reshape(n, _D // 2, 2), jnp.int32)
    src = edge_index_labeled[0]
    dst = edge_index_labeled[1]
    pred = _edge_dot(h32, src, dst)
    return (pred, edge_label)
